# baseline (speedup 1.0000x reference)
import functools

import jax
import jax.numpy as jnp
from jax import lax
from jax.experimental import pallas as pl
from jax.experimental.pallas import tpu as pltpu
from jax.experimental.pallas import tpu_sc as plsc

_N = 10000
_E = 320000
_D = 128
_G = 64
_NP = 10240
_NPH = _NP // 2
_TSTRIPE = _NPH // 16
_CCH = 64
_CHUNK = 128
_NW = 32
_CPT = 80
_EPT = _CPT * _CHUNK
_EPAD = _NW * _EPT
_CPT2 = 2 * _CPT
_EPT2 = 2 * _EPT

_MESH = plsc.VectorSubcoreMesh(core_axis_name="c", subcore_axis_name="s")


def _fill2d(ref, rows, cols, value):
    def row(i, _):
        def col(k, _):
            ref[i, pl.ds(k * 16, 16)] = jnp.full((16,), value, jnp.float32)
            return 0
        return lax.fori_loop(0, cols // 16, col, 0)
    lax.fori_loop(0, rows, row, 0)


def _zero_acc(zbuf, acc, s):
    row0 = s * _TSTRIPE
    for k in range(_TSTRIPE // _CCH):
        pltpu.sync_copy(zbuf.at[pl.ds(0, _CCH)],
                        acc.at[pl.ds(row0 + k * _CCH, _CCH)])


def _copy_out(acc, stage, out_hbm, c, s):
    row0 = s * _TSTRIPE
    for k in range(_TSTRIPE // _CCH):
        r = row0 + k * _CCH
        pltpu.sync_copy(acc.at[pl.ds(r, _CCH)], stage.at[pl.ds(0, _CCH)])
        pltpu.sync_copy(stage.at[pl.ds(0, _CCH)],
                        out_hbm.at[pl.ds(c * _NPH + r, _CCH)])


_K = 2


@functools.partial(
    pl.kernel,
    out_type=jax.ShapeDtypeStruct((_NP, _D), jnp.float32),
    mesh=_MESH,
    scratch_types=[
        [pltpu.VMEM((_CHUNK,), jnp.int32) for _ in range(_K)],
        [pltpu.SemaphoreType.DMA for _ in range(_K)],
        pltpu.VMEM((_CHUNK, _D), jnp.float32),
        pltpu.VMEM((_CHUNK, _D), jnp.float32),
        pltpu.VMEM_SHARED((_NPH, _D), jnp.float32),
    ],
)
def _deg_pass(dst_hbm, out_hbm, dsts, dsems, ones_v, stage_v, acc):
    c = lax.axis_index("c")
    s = lax.axis_index("s")
    _fill2d(ones_v, _CHUNK, _D, 1.0)
    _fill2d(stage_v, _CHUNK, _D, 0.0)
    _zero_acc(stage_v, acc, s)
    plsc.subcore_barrier()
    ebase = c * _EPAD + s * _EPT2

    def outer(g, _):
        for b in range(_K):
            t = g * _K + b
            pltpu.make_async_copy(
                dst_hbm.at[pl.ds(ebase + t * _CHUNK, _CHUNK)],
                dsts[b], dsems[b]).wait()
            pltpu.sync_copy(
                ones_v, acc.at[plsc.Indices(dsts[b], ignored_value=-1)],
                add=True)
            tn = t + _K

            @pl.when(tn < _CPT2)
            def _():
                pltpu.async_copy(dst_hbm.at[pl.ds(ebase + tn * _CHUNK, _CHUNK)],
                                 dsts[b], dsems[b])
        return 0
    for b in range(_K):
        pltpu.async_copy(dst_hbm.at[pl.ds(ebase + b * _CHUNK, _CHUNK)],
                         dsts[b], dsems[b])
    lax.fori_loop(0, _CPT2 // _K, outer, 0)
    plsc.subcore_barrier()
    _copy_out(acc, stage_v, out_hbm, c, s)


_KE = 5
_H = 3


@functools.partial(
    pl.kernel,
    out_type=jax.ShapeDtypeStruct((_NP, _D), jnp.float32),
    mesh=_MESH,
    scratch_types=[
        [pltpu.VMEM((_CHUNK,), jnp.int32) for _ in range(_KE)],
        [pltpu.VMEM((_CHUNK,), jnp.int32) for _ in range(_KE)],
        [pltpu.VMEM((_CHUNK, _D), jnp.float32) for _ in range(_KE)],
        [pltpu.SemaphoreType.DMA for _ in range(_KE)],
        [pltpu.SemaphoreType.DMA for _ in range(_KE)],
        [pltpu.SemaphoreType.DMA for _ in range(_KE)],
        pltpu.VMEM_SHARED((_NPH, _D), jnp.float32),
    ],
)
def _edge_pass(src_hbm, dst_hbm, table_hbm, out_hbm,
               srcs, dsts, bufs, ssems, dsems, gsems, acc):
    c = lax.axis_index("c")
    s = lax.axis_index("s")
    _fill2d(bufs[0], _CHUNK, _D, 0.0)
    _zero_acc(bufs[0], acc, s)
    plsc.subcore_barrier()
    ebase = c * _EPAD + s * _EPT2

    def load(t, b):
        pltpu.async_copy(src_hbm.at[pl.ds(ebase + t * _CHUNK, _CHUNK)],
                         srcs[b], ssems[b])
        pltpu.async_copy(dst_hbm.at[pl.ds(ebase + t * _CHUNK, _CHUNK)],
                         dsts[b], dsems[b])

    def wait_load(t, b):
        pltpu.make_async_copy(src_hbm.at[pl.ds(ebase + t * _CHUNK, _CHUNK)],
                              srcs[b], ssems[b]).wait()
        pltpu.make_async_copy(dst_hbm.at[pl.ds(ebase + t * _CHUNK, _CHUNK)],
                              dsts[b], dsems[b]).wait()

    def gather(b):
        pltpu.async_copy(table_hbm.at[plsc.Indices(srcs[b], ignored_value=-1)],
                         bufs[b], gsems[b])

    for b in range(_KE):
        load(b, b)
    for b in range(_H):
        wait_load(b, b)
        gather(b)

    def outer(g, _):
        for b0 in range(_KE):
            t = g * _KE + b0
            tg = t + _H
            b2 = (b0 + _H) % _KE

            @pl.when(tg < _CPT2)
            def _():
                wait_load(tg, b2)
                gather(b2)

            pltpu.make_async_copy(
                table_hbm.at[plsc.Indices(srcs[b0], ignored_value=-1)],
                bufs[b0], gsems[b0]).wait()
            pltpu.sync_copy(
                bufs[b0], acc.at[plsc.Indices(dsts[b0], ignored_value=-1)],
                add=True)
            tn = t + _KE

            @pl.when(tn < _CPT2)
            def _():
                load(tn, b0)
        return 0
    lax.fori_loop(0, _CPT2 // _KE, outer, 0)
    plsc.subcore_barrier()
    _copy_out(acc, bufs[0], out_hbm, c, s)


def _dis(dp):
    return lax.rsqrt(dp[0:_N, 0:1] + 1.0)


def _tc1_body(x_ref, w1_ref, dp_ref, y1_ref):
    dis = _dis(dp_ref[...])
    xw = jnp.dot(x_ref[...], w1_ref[...], preferred_element_type=jnp.float32)
    y1_ref[0:_N, :] = dis * xw
    y1_ref[_N:_NP, :] = jnp.zeros((_NP - _N, _D), jnp.float32)


def _tc2_body(agg_ref, y1_ref, dp_ref, w2_ref, b1_ref, y2_ref):
    dis = _dis(dp_ref[...])
    agg = agg_ref[0:_N, :] + y1_ref[0:_N, :]
    h1 = jnp.maximum(dis * agg + b1_ref[...][None, :], 0.0)
    y2_ref[0:_N, :] = dis * jnp.dot(h1, w2_ref[...], preferred_element_type=jnp.float32)
    y2_ref[_N:_NP, :] = jnp.zeros((_NP - _N, _D), jnp.float32)


def _tc3_body(agg_ref, y2_ref, dp_ref, b2_ref, batch_ref, f1w_ref, f1b_ref,
              f2w_ref, f2b_ref, out_ref):
    dis = _dis(dp_ref[...])
    agg = agg_ref[0:_N, :] + y2_ref[0:_N, :]
    h2 = dis * agg + b2_ref[...][None, :]
    gi = lax.broadcasted_iota(jnp.int32, (_G, _N), 0)
    onehot = (batch_ref[...][None, :] == gi).astype(jnp.float32)
    ssum = jnp.dot(onehot, h2, preferred_element_type=jnp.float32)
    cnt = jnp.sum(onehot, axis=1, keepdims=True)
    p = ssum / jnp.maximum(cnt, 1.0)
    p = jnp.maximum(jnp.dot(p, f1w_ref[...], preferred_element_type=jnp.float32)
                    + f1b_ref[...][None, :], 0.0)
    out_ref[...] = (jnp.dot(p, f2w_ref[...], preferred_element_type=jnp.float32)
                    + f2b_ref[...][None, :])


_tc1 = pl.pallas_call(_tc1_body, out_shape=jax.ShapeDtypeStruct((_NP, _D), jnp.float32))
_tc2 = pl.pallas_call(_tc2_body, out_shape=jax.ShapeDtypeStruct((_NP, _D), jnp.float32))
_tc3 = pl.pallas_call(_tc3_body, out_shape=jax.ShapeDtypeStruct((_G, _D), jnp.float32))


def kernel(x, edge_index, batch, W1, b1, W2, b2, fc1_W, fc1_b, fc2_W, fc2_b):
    pad = jnp.full((_EPAD - _E,), _N, jnp.int32)
    src = jnp.concatenate([edge_index[0], pad])
    dst = jnp.concatenate([edge_index[1], pad])
    lo = dst < _NPH
    dstg = jnp.concatenate([jnp.where(lo, dst, -1),
                            jnp.where(lo, -1, dst - _NPH)])
    srcg = jnp.concatenate([jnp.where(lo, src, -1),
                            jnp.where(lo, -1, src)])

    dp = _deg_pass(dstg)
    y1 = _tc1(x, W1, dp)
    agg1 = _edge_pass(srcg, dstg, y1)
    y2 = _tc2(agg1, y1, dp, W2, b1)
    agg2 = _edge_pass(srcg, dstg, y2)
    return _tc3(agg2, y2, dp, b2, batch, fc1_W, fc1_b, fc2_W, fc2_b)


# speedup vs baseline: 1.1328x; 1.1328x over previous
import functools

import jax
import jax.numpy as jnp
from jax import lax
from jax.experimental import pallas as pl
from jax.experimental.pallas import tpu as pltpu
from jax.experimental.pallas import tpu_sc as plsc

_N = 10000
_E = 320000
_D = 128
_G = 64
_NP = 10240
_STRIPE = _NP // 16
_NW = 32
_EPAD = 327680

_DC = 128
_DCPT = 80
_DK = 2

_EC = 80
_ECPT = 128
_KE = 4
_H = 3
_KL = 8

_MESH = plsc.VectorSubcoreMesh(core_axis_name="c", subcore_axis_name="s")


def _fill2d(ref, rows, cols, value):
    def row(i, _):
        def col(k, _):
            ref[i, pl.ds(k * 16, 16)] = jnp.full((16,), value, jnp.float32)
            return 0
        return lax.fori_loop(0, cols // 16, col, 0)
    lax.fori_loop(0, rows, row, 0)


def _zero_acc(zbuf, acc, s, rows):
    row0 = s * _STRIPE
    for k in range(_STRIPE // rows):
        pltpu.sync_copy(zbuf.at[pl.ds(0, rows)],
                        acc.at[pl.ds(row0 + k * rows, rows)])


def _copy_out(acc, stage, out_hbm, c, s, rows):
    row0 = s * _STRIPE
    for k in range(_STRIPE // rows):
        r = row0 + k * rows
        pltpu.sync_copy(acc.at[pl.ds(r, rows)], stage.at[pl.ds(0, rows)])
        pltpu.sync_copy(stage.at[pl.ds(0, rows)],
                        out_hbm.at[pl.ds(c * _NP + r, rows)])


@functools.partial(
    pl.kernel,
    out_type=jax.ShapeDtypeStruct((2 * _NP, _D), jnp.float32),
    mesh=_MESH,
    scratch_types=[
        [pltpu.VMEM((_DC,), jnp.int32) for _ in range(_DK)],
        [pltpu.SemaphoreType.DMA for _ in range(_DK)],
        pltpu.VMEM((_DC, _D), jnp.float32),
        pltpu.VMEM((_DC, _D), jnp.float32),
        pltpu.VMEM_SHARED((_NP, _D), jnp.float32),
    ],
)
def _deg_pass(dst_hbm, out_hbm, dsts, dsems, ones_v, stage_v, acc):
    c = lax.axis_index("c")
    s = lax.axis_index("s")
    _fill2d(ones_v, _DC, _D, 1.0)
    _fill2d(stage_v, _DC, _D, 0.0)
    _zero_acc(stage_v, acc, s, _DC)
    plsc.subcore_barrier()
    ebase = (c * 16 + s) * _DCPT * _DC
    for b in range(_DK):
        pltpu.async_copy(dst_hbm.at[pl.ds(ebase + b * _DC, _DC)],
                         dsts[b], dsems[b])

    def outer(g, _):
        for b in range(_DK):
            t = g * _DK + b
            pltpu.make_async_copy(
                dst_hbm.at[pl.ds(ebase + t * _DC, _DC)],
                dsts[b], dsems[b]).wait()
            pltpu.sync_copy(ones_v, acc.at[dsts[b]], add=True)
            tn = t + _DK

            @pl.when(tn < _DCPT)
            def _():
                pltpu.async_copy(dst_hbm.at[pl.ds(ebase + tn * _DC, _DC)],
                                 dsts[b], dsems[b])
        return 0
    lax.fori_loop(0, _DCPT // _DK, outer, 0)
    plsc.subcore_barrier()
    _copy_out(acc, stage_v, out_hbm, c, s, _DC)


@functools.partial(
    pl.kernel,
    out_type=jax.ShapeDtypeStruct((2 * _NP, _D), jnp.float32),
    mesh=_MESH,
    scratch_types=[
        [pltpu.VMEM((_EC,), jnp.int32) for _ in range(_KL)],
        [pltpu.VMEM((_EC,), jnp.int32) for _ in range(_KL)],
        [pltpu.VMEM((_EC, _D), jnp.float32) for _ in range(_KE)],
        [pltpu.SemaphoreType.DMA for _ in range(_KL)],
        [pltpu.SemaphoreType.DMA for _ in range(_KL)],
        [pltpu.SemaphoreType.DMA for _ in range(_KE)],
        pltpu.VMEM_SHARED((_NP, _D), jnp.float32),
    ],
)
def _edge_pass(src_hbm, dst_hbm, table_hbm, out_hbm,
               srcs, dsts, bufs, ssems, dsems, gsems, acc):
    c = lax.axis_index("c")
    s = lax.axis_index("s")
    _fill2d(bufs[0], 64, _D, 0.0)
    _zero_acc(bufs[0], acc, s, 64)
    plsc.subcore_barrier()
    ebase = (c * 16 + s) * _ECPT * _EC

    def load(t, bl):
        pltpu.async_copy(src_hbm.at[pl.ds(ebase + t * _EC, _EC)],
                         srcs[bl], ssems[bl])
        pltpu.async_copy(dst_hbm.at[pl.ds(ebase + t * _EC, _EC)],
                         dsts[bl], dsems[bl])

    def wait_src(t, bl):
        pltpu.make_async_copy(src_hbm.at[pl.ds(ebase + t * _EC, _EC)],
                              srcs[bl], ssems[bl]).wait()

    def wait_dst(t, bl):
        pltpu.make_async_copy(dst_hbm.at[pl.ds(ebase + t * _EC, _EC)],
                              dsts[bl], dsems[bl]).wait()

    def gather(bl, bg):
        pltpu.async_copy(table_hbm.at[srcs[bl]], bufs[bg], gsems[bg])

    for t in range(_KL):
        load(t, t)
    for t in range(_H):
        wait_src(t, t)
        gather(t, t % _KE)

    def outer(g, _):
        for b0 in range(_KL):
            t = g * _KL + b0
            bg = b0 % _KE
            tg = t + _H
            blg = (b0 + _H) % _KL
            bgg = (b0 + _H) % _KE

            @pl.when(tg < _ECPT)
            def _():
                wait_src(tg, blg)
                gather(blg, bgg)

            pltpu.make_async_copy(table_hbm.at[srcs[b0]], bufs[bg],
                                  gsems[bg]).wait()
            wait_dst(t, b0)
            pltpu.sync_copy(bufs[bg], acc.at[dsts[b0]], add=True)
            tn = t + _KL

            @pl.when(tn < _ECPT)
            def _():
                load(tn, b0)
        return 0
    lax.fori_loop(0, _ECPT // _KL, outer, 0)
    plsc.subcore_barrier()
    _copy_out(acc, bufs[0], out_hbm, c, s, 64)


def _dis(dp):
    deg = dp[0:_N, 0:1] + dp[_NP:_NP + _N, 0:1] + 1.0
    return lax.rsqrt(deg)


def _tc1_body(x_ref, w1_ref, dp_ref, y1_ref):
    dis = _dis(dp_ref[...])
    xw = jnp.dot(x_ref[...], w1_ref[...], preferred_element_type=jnp.float32)
    y1_ref[0:_N, :] = dis * xw
    y1_ref[_N:_NP, :] = jnp.zeros((_NP - _N, _D), jnp.float32)


def _tc2_body(agg_ref, y1_ref, dp_ref, w2_ref, b1_ref, y2_ref):
    dis = _dis(dp_ref[...])
    agg = agg_ref[0:_N, :] + agg_ref[_NP:_NP + _N, :] + y1_ref[0:_N, :]
    h1 = jnp.maximum(dis * agg + b1_ref[...][None, :], 0.0)
    y2_ref[0:_N, :] = dis * jnp.dot(h1, w2_ref[...], preferred_element_type=jnp.float32)
    y2_ref[_N:_NP, :] = jnp.zeros((_NP - _N, _D), jnp.float32)


def _tc3_body(agg_ref, y2_ref, dp_ref, b2_ref, batch_ref, f1w_ref, f1b_ref,
              f2w_ref, f2b_ref, out_ref):
    dis = _dis(dp_ref[...])
    agg = agg_ref[0:_N, :] + agg_ref[_NP:_NP + _N, :] + y2_ref[0:_N, :]
    h2 = dis * agg + b2_ref[...][None, :]
    gi = lax.broadcasted_iota(jnp.int32, (_G, _N), 0)
    onehot = (batch_ref[...][None, :] == gi).astype(jnp.float32)
    ssum = jnp.dot(onehot, h2, preferred_element_type=jnp.float32)
    cnt = jnp.sum(onehot, axis=1, keepdims=True)
    p = ssum / jnp.maximum(cnt, 1.0)
    p = jnp.maximum(jnp.dot(p, f1w_ref[...], preferred_element_type=jnp.float32)
                    + f1b_ref[...][None, :], 0.0)
    out_ref[...] = (jnp.dot(p, f2w_ref[...], preferred_element_type=jnp.float32)
                    + f2b_ref[...][None, :])


_tc1 = pl.pallas_call(_tc1_body, out_shape=jax.ShapeDtypeStruct((_NP, _D), jnp.float32))
_tc2 = pl.pallas_call(_tc2_body, out_shape=jax.ShapeDtypeStruct((_NP, _D), jnp.float32))
_tc3 = pl.pallas_call(_tc3_body, out_shape=jax.ShapeDtypeStruct((_G, _D), jnp.float32))


def kernel(x, edge_index, batch, W1, b1, W2, b2, fc1_W, fc1_b, fc2_W, fc2_b):
    pad = jnp.full((_EPAD - _E,), _N, jnp.int32)
    src = jnp.concatenate([edge_index[0], pad])
    dst = jnp.concatenate([edge_index[1], pad])

    dp = _deg_pass(dst)
    y1 = _tc1(x, W1, dp)
    agg1 = _edge_pass(src, dst, y1)
    y2 = _tc2(agg1, y1, dp, W2, b1)
    agg2 = _edge_pass(src, dst, y2)
    return _tc3(agg2, y2, dp, b2, batch, fc1_W, fc1_b, fc2_W, fc2_b)
